# Initial kernel scaffold; baseline (speedup 1.0000x reference)
import functools
import math

import jax
import jax.numpy as jnp
from jax import lax
from jax.experimental import pallas as pl
from jax.experimental.pallas import tpu as pltpu
from jax.experimental.pallas import tpu_sc as plsc

N = 10000
D = 128
E = 320000
ALPHA = 0.1
MIX_B = math.log1p(0.5 / 3.0)

NC = 2
NS = 16
NW = NC * NS
L = 16

EPW = E // NW
EPC = E // NC
RPS = N // NS
C = 80
CHUNKS = EPW // C

_mesh = plsc.VectorSubcoreMesh(core_axis_name="c", subcore_axis_name="s")


@functools.partial(
    pl.kernel,
    out_type=jax.ShapeDtypeStruct((NW, N), jnp.float32),
    mesh=_mesh,
    scratch_types=[
        pltpu.VMEM((EPW,), jnp.int32),
        pltpu.VMEM((N,), jnp.float32),
    ],
)
def _deg_kernel(dst_hbm, deg_out, dstbuf, degbuf):
    cid = lax.axis_index("c")
    sid = lax.axis_index("s")
    wid = sid * NC + cid

    zeros = jnp.zeros((L,), jnp.float32)
    ones = jnp.ones((L,), jnp.float32)

    def _zero(i):
        degbuf[pl.ds(i * L, L)] = zeros

    pl.loop(0, N // L)(_zero)

    pltpu.sync_copy(dst_hbm.at[pl.ds(wid * EPW, EPW)], dstbuf)

    def _count(i):
        idx = dstbuf[pl.ds(i * L, L)]
        plsc.addupdate_scatter(degbuf, [idx], ones)

    pl.loop(0, EPW // L)(_count)

    pltpu.sync_copy(degbuf, deg_out.at[wid])


BN = 1000


def _scale_body(deg_ref, f_ref, g_ref):
    deg = jnp.sum(deg_ref[...], axis=0) + 1.0
    dinv = lax.rsqrt(deg)
    g_ref[...] = f_ref[...] * dinv[:, None]


_scale_kernel = pl.pallas_call(
    _scale_body,
    grid=(N // BN,),
    in_specs=[
        pl.BlockSpec((NW, BN), lambda j: (0, j)),
        pl.BlockSpec((BN, D), lambda j: (j, 0)),
    ],
    out_specs=pl.BlockSpec((BN, D), lambda j: (j, 0)),
    out_shape=jax.ShapeDtypeStruct((N, D), jnp.float32),
)


@functools.partial(
    pl.kernel,
    out_type=jax.ShapeDtypeStruct((NC, N, D), jnp.float32),
    mesh=_mesh,
    scratch_types=[
        pltpu.VMEM_SHARED((N, D), jnp.float32),
        pltpu.VMEM((C,), jnp.int32),
        pltpu.VMEM((C,), jnp.int32),
        pltpu.VMEM((C, D), jnp.float32),
        pltpu.SemaphoreType.DMA,
    ],
)
def _agg_kernel(src_hbm, dst_hbm, g_hbm, z_hbm, s_out, acc, srcbuf, dstbuf,
                rows, sem):
    cid = lax.axis_index("c")
    sid = lax.axis_index("s")

    pltpu.sync_copy(z_hbm.at[pl.ds(sid * RPS, RPS)],
                    acc.at[pl.ds(sid * RPS, RPS)])
    plsc.subcore_barrier()

    base = cid * EPC + sid * EPW

    def _step(t):
        off = base + t * C
        pltpu.sync_copy(src_hbm.at[pl.ds(off, C)], srcbuf)
        pltpu.sync_copy(dst_hbm.at[pl.ds(off, C)], dstbuf)
        pltpu.async_copy(g_hbm.at[srcbuf], rows, sem).wait()
        pltpu.sync_copy(rows, acc.at[dstbuf], add=True)

    pl.loop(0, CHUNKS)(_step)

    plsc.subcore_barrier()
    pltpu.sync_copy(acc.at[pl.ds(sid * RPS, RPS)],
                    s_out.at[cid, pl.ds(sid * RPS, RPS)])


def _final_body(deg_ref, s_ref, g_ref, h_ref, w_ref, o_ref):
    deg = jnp.sum(deg_ref[...], axis=0) + 1.0
    dinv = lax.rsqrt(deg)
    s = s_ref[0] + s_ref[1] + g_ref[...]
    tr = (1.0 - ALPHA) * (s * dinv[:, None]) + ALPHA * h_ref[...]
    o_ref[...] = (1.0 - MIX_B) * tr + MIX_B * jnp.dot(
        tr, w_ref[...], preferred_element_type=jnp.float32)


_final_kernel = pl.pallas_call(
    _final_body,
    grid=(N // BN,),
    in_specs=[
        pl.BlockSpec((NW, BN), lambda j: (0, j)),
        pl.BlockSpec((NC, BN, D), lambda j: (0, j, 0)),
        pl.BlockSpec((BN, D), lambda j: (j, 0)),
        pl.BlockSpec((BN, D), lambda j: (j, 0)),
        pl.BlockSpec((D, D), lambda j: (0, 0)),
    ],
    out_specs=pl.BlockSpec((BN, D), lambda j: (j, 0)),
    out_shape=jax.ShapeDtypeStruct((N, D), jnp.float32),
)


def kernel(features, H0, W, edge_index):
    src = edge_index[0]
    dst = edge_index[1]
    deg_p = _deg_kernel(dst)
    g = _scale_kernel(deg_p, features)
    zeros = jnp.zeros((N, D), jnp.float32)
    s_p = _agg_kernel(src, dst, g, zeros)
    return _final_kernel(deg_p, s_p, g, H0, W)


# speedup vs baseline: 15.7347x; 15.7347x over previous
import functools
import math

import jax
import jax.numpy as jnp
from jax import lax
from jax.experimental import pallas as pl
from jax.experimental.pallas import tpu as pltpu
from jax.experimental.pallas import tpu_sc as plsc

N = 10000
D = 128
E = 320000
ALPHA = 0.1
MIX_B = math.log1p(0.5 / 3.0)

NC = 2
NS = 16
NW = NC * NS
L = 16

EPW = E // NW
EPC = E // NC
RPS = N // NS
RCHUNK = 80
C = 80
CHUNKS = EPW // C

_mesh = plsc.VectorSubcoreMesh(core_axis_name="c", subcore_axis_name="s")


@functools.partial(
    pl.kernel,
    out_type=jax.ShapeDtypeStruct((NW * N,), jnp.float32),
    mesh=_mesh,
    scratch_types=[
        pltpu.VMEM((EPW,), jnp.int32),
        pltpu.VMEM((N,), jnp.float32),
    ],
    compiler_params=pltpu.CompilerParams(needs_layout_passes=False),
)
def _deg_kernel(dst_hbm, deg_out, dstbuf, degbuf):
    cid = lax.axis_index("c")
    sid = lax.axis_index("s")
    wid = sid * NC + cid

    zeros = jnp.zeros((L,), jnp.float32)
    ones = jnp.ones((L,), jnp.float32)

    def _zero(i):
        degbuf[pl.ds(i * L, L)] = zeros

    pl.loop(0, N // L)(_zero)

    pltpu.sync_copy(dst_hbm.at[pl.ds(wid * EPW, EPW)], dstbuf)

    def _count(i):
        idx = dstbuf[pl.ds(i * L, L)]
        plsc.addupdate_scatter(degbuf, [idx], ones)

    pl.loop(0, EPW // L)(_count)

    pltpu.sync_copy(degbuf, deg_out.at[pl.ds(wid * N, N)])


BN = 1000


def _scale_body(deg_ref, f_ref, g_ref):
    deg = jnp.sum(deg_ref[0], axis=0) + 1.0
    dinv = lax.rsqrt(deg)
    g_ref[...] = f_ref[...] * dinv[:, None]


_scale_kernel = pl.pallas_call(
    _scale_body,
    grid=(N // BN,),
    in_specs=[
        pl.BlockSpec((1, NW, BN), lambda j: (j, 0, 0)),
        pl.BlockSpec((BN, D), lambda j: (j, 0)),
    ],
    out_specs=pl.BlockSpec((BN, D), lambda j: (j, 0)),
    out_shape=jax.ShapeDtypeStruct((N, D), jnp.float32),
)


@functools.partial(
    pl.kernel,
    out_type=jax.ShapeDtypeStruct((NC, N, D), jnp.float32),
    mesh=_mesh,
    scratch_types=[
        pltpu.VMEM_SHARED((N, D), jnp.float32),
        pltpu.VMEM((C,), jnp.int32),
        pltpu.VMEM((C,), jnp.int32),
        pltpu.VMEM((C, D), jnp.float32),
        pltpu.SemaphoreType.DMA,
    ],
)
def _agg_kernel(src_hbm, dst_hbm, g_hbm, z_hbm, s_out, acc, srcbuf, dstbuf,
                rows, sem):
    cid = lax.axis_index("c")
    sid = lax.axis_index("s")

    def _zero(c):
        pltpu.sync_copy(z_hbm.at[pl.ds(c * RCHUNK, RCHUNK)],
                        acc.at[pl.ds(c * RCHUNK, RCHUNK)])

    pl.loop(sid, N // RCHUNK, step=NS)(_zero)
    plsc.subcore_barrier()

    base = cid * EPC + sid * EPW

    def _step(t):
        off = base + t * C
        pltpu.sync_copy(src_hbm.at[pl.ds(off, C)], srcbuf)
        pltpu.sync_copy(dst_hbm.at[pl.ds(off, C)], dstbuf)
        pltpu.async_copy(g_hbm.at[srcbuf], rows, sem).wait()
        pltpu.sync_copy(rows, acc.at[dstbuf], add=True)

    pl.loop(0, CHUNKS)(_step)

    plsc.subcore_barrier()

    def _dump(c):
        pltpu.sync_copy(acc.at[pl.ds(c * RCHUNK, RCHUNK)],
                        s_out.at[cid, pl.ds(c * RCHUNK, RCHUNK)])

    pl.loop(sid, N // RCHUNK, step=NS)(_dump)


def _final_body(deg_ref, s_ref, g_ref, h_ref, w_ref, o_ref):
    deg = jnp.sum(deg_ref[0], axis=0) + 1.0
    dinv = lax.rsqrt(deg)
    s = s_ref[0] + s_ref[1] + g_ref[...]
    tr = (1.0 - ALPHA) * (s * dinv[:, None]) + ALPHA * h_ref[...]
    o_ref[...] = (1.0 - MIX_B) * tr + MIX_B * jnp.dot(
        tr, w_ref[...], preferred_element_type=jnp.float32)


_final_kernel = pl.pallas_call(
    _final_body,
    grid=(N // BN,),
    in_specs=[
        pl.BlockSpec((1, NW, BN), lambda j: (j, 0, 0)),
        pl.BlockSpec((NC, BN, D), lambda j: (0, j, 0)),
        pl.BlockSpec((BN, D), lambda j: (j, 0)),
        pl.BlockSpec((BN, D), lambda j: (j, 0)),
        pl.BlockSpec((D, D), lambda j: (0, 0)),
    ],
    out_specs=pl.BlockSpec((BN, D), lambda j: (j, 0)),
    out_shape=jax.ShapeDtypeStruct((N, D), jnp.float32),
)


def kernel(features, H0, W, edge_index):
    src = edge_index[0]
    dst = edge_index[1]
    deg_p = _deg_kernel(dst)
    deg_t = deg_p.reshape(NW, N // BN, BN).transpose(1, 0, 2)
    g = _scale_kernel(deg_t, features)
    zeros = jnp.zeros((N, D), jnp.float32)
    s_p = _agg_kernel(src, dst, g, zeros)
    return _final_kernel(deg_t, s_p, g, H0, W)


# speedup vs baseline: 26.4505x; 1.6810x over previous
import functools
import math

import jax
import jax.numpy as jnp
from jax import lax
from jax.experimental import pallas as pl
from jax.experimental.pallas import tpu as pltpu
from jax.experimental.pallas import tpu_sc as plsc

N = 10000
D = 128
E = 320000
ALPHA = 0.1
MIX_B = math.log1p(0.5 / 3.0)

NC = 2
NS = 16
NW = NC * NS
L = 16

EPW = E // NW
EPC = E // NC
RPS = N // NS
RCHUNK = 80
C = 80
CHUNKS = EPW // C

_mesh = plsc.VectorSubcoreMesh(core_axis_name="c", subcore_axis_name="s")


@functools.partial(
    pl.kernel,
    out_type=jax.ShapeDtypeStruct((NW * N,), jnp.float32),
    mesh=_mesh,
    scratch_types=[
        pltpu.VMEM((EPW,), jnp.int32),
        pltpu.VMEM((N,), jnp.float32),
    ],
    compiler_params=pltpu.CompilerParams(needs_layout_passes=False),
)
def _deg_kernel(dst_hbm, deg_out, dstbuf, degbuf):
    cid = lax.axis_index("c")
    sid = lax.axis_index("s")
    wid = sid * NC + cid

    zeros = jnp.zeros((L,), jnp.float32)
    ones = jnp.ones((L,), jnp.float32)

    def _zero(i):
        degbuf[pl.ds(i * L, L)] = zeros

    pl.loop(0, N // L)(_zero)

    pltpu.sync_copy(dst_hbm.at[pl.ds(wid * EPW, EPW)], dstbuf)

    def _count(i):
        idx = dstbuf[pl.ds(i * L, L)]
        plsc.addupdate_scatter(degbuf, [idx], ones)

    pl.loop(0, EPW // L)(_count)

    pltpu.sync_copy(degbuf, deg_out.at[pl.ds(wid * N, N)])


BN = 1000


def _scale_body(deg_ref, f_ref, g_ref):
    deg = jnp.sum(deg_ref[0], axis=0) + 1.0
    dinv = lax.rsqrt(deg)
    g_ref[...] = f_ref[...] * dinv[:, None]


_scale_kernel = pl.pallas_call(
    _scale_body,
    grid=(N // BN,),
    in_specs=[
        pl.BlockSpec((1, NW, BN), lambda j: (j, 0, 0)),
        pl.BlockSpec((BN, D), lambda j: (j, 0)),
    ],
    out_specs=pl.BlockSpec((BN, D), lambda j: (j, 0)),
    out_shape=jax.ShapeDtypeStruct((N, D), jnp.float32),
)


@functools.partial(
    pl.kernel,
    out_type=jax.ShapeDtypeStruct((NC, N, D), jnp.float32),
    mesh=_mesh,
    scratch_types=[
        pltpu.VMEM_SHARED((N, D), jnp.float32),
        pltpu.VMEM((3, C), jnp.int32),
        pltpu.VMEM((3, C), jnp.int32),
        pltpu.VMEM((2 * C, D), jnp.float32),
        pltpu.SemaphoreType.DMA,
        pltpu.SemaphoreType.DMA,
        pltpu.SemaphoreType.DMA,
    ],
)
def _agg_kernel(src_hbm, dst_hbm, g_hbm, z_hbm, s_out, acc, srcb, dstb,
                rows, gsem, ssem, dsem):
    cid = lax.axis_index("c")
    sid = lax.axis_index("s")
    wid = cid * NS + sid

    def _zero(c):
        pltpu.sync_copy(z_hbm.at[pl.ds(c * RCHUNK, RCHUNK)],
                        acc.at[pl.ds(c * RCHUNK, RCHUNK)])

    pl.loop(sid, N // RCHUNK, step=NS)(_zero)

    def _idx_start(t):
        s = t % 3
        pltpu.async_copy(src_hbm.at[wid, t], srcb.at[s], ssem)
        pltpu.async_copy(dst_hbm.at[wid, t], dstb.at[s], dsem)

    def _idx_wait(t):
        s = t % 3
        pltpu.make_async_copy(src_hbm.at[0, 0], srcb.at[s], ssem).wait()
        pltpu.make_async_copy(dst_hbm.at[0, 0], dstb.at[s], dsem).wait()

    _idx_start(0)
    plsc.subcore_barrier()
    _idx_wait(0)
    pltpu.async_copy(g_hbm.at[srcb.at[0]], rows.at[pl.ds(0, C)], gsem)
    _idx_start(1)

    def _step(t):
        b = (t % 2) * C
        nb = ((t + 1) % 2) * C
        rows_cur = rows.at[pl.ds(b, C)]
        pltpu.make_async_copy(g_hbm.at[pl.ds(0, C)], rows_cur, gsem).wait()

        @pl.when(t + 1 < CHUNKS)
        def _():
            _idx_wait(t + 1)
            pltpu.async_copy(g_hbm.at[srcb.at[(t + 1) % 3]],
                             rows.at[pl.ds(nb, C)], gsem)

        @pl.when(t + 2 < CHUNKS)
        def _():
            _idx_start(t + 2)

        pltpu.sync_copy(rows_cur, acc.at[dstb.at[t % 3]], add=True)

    pl.loop(0, CHUNKS)(_step)

    plsc.subcore_barrier()

    def _dump(c):
        pltpu.sync_copy(acc.at[pl.ds(c * RCHUNK, RCHUNK)],
                        s_out.at[cid, pl.ds(c * RCHUNK, RCHUNK)])

    pl.loop(sid, N // RCHUNK, step=NS)(_dump)


def _final_body(deg_ref, s_ref, g_ref, h_ref, w_ref, o_ref):
    deg = jnp.sum(deg_ref[0], axis=0) + 1.0
    dinv = lax.rsqrt(deg)
    s = s_ref[0] + s_ref[1] + g_ref[...]
    tr = (1.0 - ALPHA) * (s * dinv[:, None]) + ALPHA * h_ref[...]
    o_ref[...] = (1.0 - MIX_B) * tr + MIX_B * jnp.dot(
        tr, w_ref[...], preferred_element_type=jnp.float32)


_final_kernel = pl.pallas_call(
    _final_body,
    grid=(N // BN,),
    in_specs=[
        pl.BlockSpec((1, NW, BN), lambda j: (j, 0, 0)),
        pl.BlockSpec((NC, BN, D), lambda j: (0, j, 0)),
        pl.BlockSpec((BN, D), lambda j: (j, 0)),
        pl.BlockSpec((BN, D), lambda j: (j, 0)),
        pl.BlockSpec((D, D), lambda j: (0, 0)),
    ],
    out_specs=pl.BlockSpec((BN, D), lambda j: (j, 0)),
    out_shape=jax.ShapeDtypeStruct((N, D), jnp.float32),
)


def kernel(features, H0, W, edge_index):
    src = edge_index[0]
    dst = edge_index[1]
    deg_p = _deg_kernel(dst)
    deg_t = deg_p.reshape(NW, N // BN, BN).transpose(1, 0, 2)
    g = _scale_kernel(deg_t, features)
    zeros = jnp.zeros((N, D), jnp.float32)
    src3 = src.reshape(NW, CHUNKS, C)
    dst3 = dst.reshape(NW, CHUNKS, C)
    s_p = _agg_kernel(src3, dst3, g, zeros)
    return _final_kernel(deg_t, s_p, g, H0, W)


# speedup vs baseline: 27.2677x; 1.0309x over previous
import functools
import math

import jax
import jax.numpy as jnp
from jax import lax
from jax.experimental import pallas as pl
from jax.experimental.pallas import tpu as pltpu
from jax.experimental.pallas import tpu_sc as plsc

N = 10000
D = 128
E = 320000
ALPHA = 0.1
MIX_B = math.log1p(0.5 / 3.0)

NC = 2
NS = 16
NW = NC * NS
L = 16

EPW = E // NW
EPC = E // NC
RPS = N // NS
RCHUNK = 80
C = 80
CHUNKS = EPW // C
BN = 1000

_mesh = plsc.VectorSubcoreMesh(core_axis_name="c", subcore_axis_name="s")


@functools.partial(
    pl.kernel,
    out_type=jax.ShapeDtypeStruct((NW * N,), jnp.float32),
    mesh=_mesh,
    scratch_types=[
        pltpu.VMEM((EPW,), jnp.int32),
        pltpu.VMEM((N,), jnp.float32),
    ],
    compiler_params=pltpu.CompilerParams(needs_layout_passes=False),
)
def _deg_kernel(dst_hbm, deg_out, dstbuf, degbuf):
    cid = lax.axis_index("c")
    sid = lax.axis_index("s")
    wid = sid * NC + cid

    zeros = jnp.zeros((L,), jnp.float32)
    ones = jnp.ones((L,), jnp.float32)

    def _zero(i):
        degbuf[pl.ds(i * L, L)] = zeros

    pl.loop(0, N // L)(_zero)

    pltpu.sync_copy(dst_hbm.at[pl.ds(wid * EPW, EPW)], dstbuf)

    def _count(i):
        idx = dstbuf[pl.ds(i * L, L)]
        plsc.addupdate_scatter(degbuf, [idx], ones)

    pl.loop(0, EPW // L)(_count)

    pltpu.sync_copy(degbuf, deg_out.at[pl.ds(wid * N, N)])


def _scale_body(deg_ref, f_ref, g_ref):
    deg = jnp.sum(deg_ref[0], axis=0) + 1.0
    dinv = lax.rsqrt(deg)
    g_ref[...] = f_ref[...] * dinv[:, None]


_scale_kernel = pl.pallas_call(
    _scale_body,
    grid=(N // BN,),
    in_specs=[
        pl.BlockSpec((1, NW, BN), lambda j: (j, 0, 0)),
        pl.BlockSpec((BN, D), lambda j: (j, 0)),
    ],
    out_specs=pl.BlockSpec((BN, D), lambda j: (j, 0)),
    out_shape=jax.ShapeDtypeStruct((N, D), jnp.float32),
)


@functools.partial(
    pl.kernel,
    out_type=jax.ShapeDtypeStruct((NC, N, D), jnp.float32),
    mesh=_mesh,
    scratch_types=[
        pltpu.VMEM_SHARED((N, D), jnp.float32),
        pltpu.VMEM((3, C), jnp.int32),
        pltpu.VMEM((3, C), jnp.int32),
        pltpu.VMEM((2 * C, D), jnp.float32),
        pltpu.SemaphoreType.DMA,
        pltpu.SemaphoreType.DMA,
        pltpu.SemaphoreType.DMA,
        pltpu.SemaphoreType.DMA,
    ],
)
def _agg_kernel(src_hbm, dst_hbm, g_hbm, s_out, acc, srcb, dstb,
                rows, gsem, ssem, dsem, asem):
    cid = lax.axis_index("c")
    sid = lax.axis_index("s")
    wid = cid * NS + sid

    zv = jnp.zeros((L,), jnp.float32)

    def _zrow(r):
        def _zcol(j):
            rows[r, pl.ds(j * L, L)] = zv

        pl.loop(0, D // L)(_zcol)

    pl.loop(0, C)(_zrow)

    def _zero(c):
        pltpu.sync_copy(rows.at[pl.ds(0, C)],
                        acc.at[pl.ds(c * RCHUNK, RCHUNK)])

    pl.loop(sid, N // RCHUNK, step=NS)(_zero)

    def _idx_start(t):
        s = t % 3
        pltpu.async_copy(src_hbm.at[wid, t], srcb.at[s], ssem)
        pltpu.async_copy(dst_hbm.at[wid, t], dstb.at[s], dsem)

    def _idx_wait(t):
        s = t % 3
        pltpu.make_async_copy(src_hbm.at[0, 0], srcb.at[s], ssem).wait()
        pltpu.make_async_copy(dst_hbm.at[0, 0], dstb.at[s], dsem).wait()

    _idx_start(0)
    plsc.subcore_barrier()
    _idx_wait(0)
    pltpu.async_copy(g_hbm.at[srcb.at[0]], rows.at[pl.ds(0, C)], gsem)
    _idx_start(1)

    def _step(t):
        b = (t % 2) * C
        nb = ((t + 1) % 2) * C
        rows_cur = rows.at[pl.ds(b, C)]
        rows_nxt = rows.at[pl.ds(nb, C)]
        pltpu.make_async_copy(g_hbm.at[pl.ds(0, C)], rows_cur, gsem).wait()

        @pl.when(t > 0)
        def _():
            pltpu.make_async_copy(rows_nxt, acc.at[dstb.at[(t + 2) % 3]],
                                  asem).wait()

        @pl.when(t + 1 < CHUNKS)
        def _():
            _idx_wait(t + 1)
            pltpu.async_copy(g_hbm.at[srcb.at[(t + 1) % 3]], rows_nxt, gsem)

        @pl.when(t + 2 < CHUNKS)
        def _():
            _idx_start(t + 2)

        pltpu.async_copy(rows_cur, acc.at[dstb.at[t % 3]], asem, add=True)

    pl.loop(0, CHUNKS)(_step)

    pltpu.make_async_copy(rows.at[pl.ds(((CHUNKS - 1) % 2) * C, C)],
                          acc.at[dstb.at[(CHUNKS - 1) % 3]], asem).wait()
    plsc.subcore_barrier()

    def _dump(c):
        pltpu.sync_copy(acc.at[pl.ds(c * RCHUNK, RCHUNK)],
                        s_out.at[cid, pl.ds(c * RCHUNK, RCHUNK)])

    pl.loop(sid, N // RCHUNK, step=NS)(_dump)


def _final_body(deg_ref, s_ref, g_ref, h_ref, w_ref, o_ref):
    deg = jnp.sum(deg_ref[0], axis=0) + 1.0
    dinv = lax.rsqrt(deg)
    s = s_ref[0] + s_ref[1] + g_ref[...]
    tr = (1.0 - ALPHA) * (s * dinv[:, None]) + ALPHA * h_ref[...]
    o_ref[...] = (1.0 - MIX_B) * tr + MIX_B * jnp.dot(
        tr, w_ref[...], preferred_element_type=jnp.float32)


_final_kernel = pl.pallas_call(
    _final_body,
    grid=(N // BN,),
    in_specs=[
        pl.BlockSpec((1, NW, BN), lambda j: (j, 0, 0)),
        pl.BlockSpec((NC, BN, D), lambda j: (0, j, 0)),
        pl.BlockSpec((BN, D), lambda j: (j, 0)),
        pl.BlockSpec((BN, D), lambda j: (j, 0)),
        pl.BlockSpec((D, D), lambda j: (0, 0)),
    ],
    out_specs=pl.BlockSpec((BN, D), lambda j: (j, 0)),
    out_shape=jax.ShapeDtypeStruct((N, D), jnp.float32),
)


def kernel(features, H0, W, edge_index):
    src = edge_index[0]
    dst = edge_index[1]
    deg_p = _deg_kernel(dst)
    deg_t = deg_p.reshape(NW, N // BN, BN).transpose(1, 0, 2)
    g = _scale_kernel(deg_t, features)
    src3 = src.reshape(NW, CHUNKS, C)
    dst3 = dst.reshape(NW, CHUNKS, C)
    s_p = _agg_kernel(src3, dst3, g)
    return _final_kernel(deg_t, s_p, g, H0, W)


# speedup vs baseline: 28.6527x; 1.0508x over previous
import functools
import math

import jax
import jax.numpy as jnp
from jax import lax
from jax.experimental import pallas as pl
from jax.experimental.pallas import tpu as pltpu
from jax.experimental.pallas import tpu_sc as plsc

N = 10000
D = 128
E = 320000
ALPHA = 0.1
MIX_B = math.log1p(0.5 / 3.0)

NC = 2
NS = 16
NW = NC * NS
L = 16

EPW = E // NW
EPC = E // NC
RPS = N // NS
RCHUNK = 80
C = 80
CHUNKS = EPW // C
BN = 1000

_mesh = plsc.VectorSubcoreMesh(core_axis_name="c", subcore_axis_name="s")


@functools.partial(
    pl.kernel,
    out_type=jax.ShapeDtypeStruct((NW * N,), jnp.float32),
    mesh=_mesh,
    scratch_types=[
        pltpu.VMEM((EPW,), jnp.int32),
        pltpu.VMEM((N,), jnp.float32),
    ],
    compiler_params=pltpu.CompilerParams(needs_layout_passes=False),
)
def _deg_kernel(edges_hbm, deg_out, dstbuf, degbuf):
    cid = lax.axis_index("c")
    sid = lax.axis_index("s")
    wid = sid * NC + cid

    zeros = jnp.zeros((L,), jnp.float32)
    ones = jnp.ones((L,), jnp.float32)

    def _zero(i):
        degbuf[pl.ds(i * L, L)] = zeros

    pl.loop(0, N // L)(_zero)

    pltpu.sync_copy(edges_hbm.at[pl.ds(E + wid * EPW, EPW)], dstbuf)

    def _count(i):
        idx = dstbuf[pl.ds(i * L, L)]
        plsc.addupdate_scatter(degbuf, [idx], ones)

    pl.loop(0, EPW // L)(_count)

    pltpu.sync_copy(degbuf, deg_out.at[pl.ds(wid * N, N)])


def _scale_body(deg_ref, f_ref, g_ref):
    deg = jnp.sum(deg_ref[0], axis=0) + 1.0
    dinv = lax.rsqrt(deg)
    g_ref[...] = f_ref[...] * dinv[:, None]


_scale_kernel = pl.pallas_call(
    _scale_body,
    grid=(N // BN,),
    in_specs=[
        pl.BlockSpec((1, NW, BN), lambda j: (j, 0, 0)),
        pl.BlockSpec((BN, D), lambda j: (j, 0)),
    ],
    out_specs=pl.BlockSpec((BN, D), lambda j: (j, 0)),
    out_shape=jax.ShapeDtypeStruct((N, D), jnp.float32),
)


@functools.partial(
    pl.kernel,
    out_type=jax.ShapeDtypeStruct((NC, N, D), jnp.float32),
    mesh=_mesh,
    scratch_types=[
        pltpu.VMEM_SHARED((N, D), jnp.float32),
        pltpu.VMEM((3, C), jnp.int32),
        pltpu.VMEM((3, C), jnp.int32),
        pltpu.VMEM((2 * C, D), jnp.float32),
        pltpu.SemaphoreType.DMA,
        pltpu.SemaphoreType.DMA,
        pltpu.SemaphoreType.DMA,
        pltpu.SemaphoreType.DMA,
    ],
)
def _agg_kernel(edges_hbm, g_hbm, s_out, acc, srcb, dstb,
                rows, gsem, ssem, dsem, asem):
    cid = lax.axis_index("c")
    sid = lax.axis_index("s")
    wid = cid * NS + sid

    zv = jnp.zeros((L,), jnp.float32)

    def _zrow(r):
        def _zcol(j):
            rows[r, pl.ds(j * L, L)] = zv

        pl.loop(0, D // L)(_zcol)

    pl.loop(0, C)(_zrow)

    def _zero(c):
        pltpu.sync_copy(rows.at[pl.ds(0, C)],
                        acc.at[pl.ds(c * RCHUNK, RCHUNK)])

    pl.loop(sid, N // RCHUNK, step=NS)(_zero)

    def _idx_start(t):
        s = t % 3
        off = wid * EPW + t * C
        pltpu.async_copy(edges_hbm.at[pl.ds(off, C)], srcb.at[s], ssem)
        pltpu.async_copy(edges_hbm.at[pl.ds(E + off, C)], dstb.at[s], dsem)

    def _idx_wait(t):
        s = t % 3
        pltpu.make_async_copy(edges_hbm.at[pl.ds(0, C)], srcb.at[s],
                              ssem).wait()
        pltpu.make_async_copy(edges_hbm.at[pl.ds(0, C)], dstb.at[s],
                              dsem).wait()

    _idx_start(0)
    plsc.subcore_barrier()
    _idx_wait(0)
    pltpu.async_copy(g_hbm.at[srcb.at[0]], rows.at[pl.ds(0, C)], gsem)
    _idx_start(1)

    def _step(t):
        b = (t % 2) * C
        nb = ((t + 1) % 2) * C
        rows_cur = rows.at[pl.ds(b, C)]
        rows_nxt = rows.at[pl.ds(nb, C)]
        pltpu.make_async_copy(g_hbm.at[pl.ds(0, C)], rows_cur, gsem).wait()

        @pl.when(t > 0)
        def _():
            pltpu.make_async_copy(rows_nxt, acc.at[dstb.at[(t + 2) % 3]],
                                  asem).wait()

        @pl.when(t + 1 < CHUNKS)
        def _():
            _idx_wait(t + 1)
            pltpu.async_copy(g_hbm.at[srcb.at[(t + 1) % 3]], rows_nxt, gsem)

        @pl.when(t + 2 < CHUNKS)
        def _():
            _idx_start(t + 2)

        pltpu.async_copy(rows_cur, acc.at[dstb.at[t % 3]], asem, add=True)

    pl.loop(0, CHUNKS)(_step)

    pltpu.make_async_copy(rows.at[pl.ds(((CHUNKS - 1) % 2) * C, C)],
                          acc.at[dstb.at[(CHUNKS - 1) % 3]], asem).wait()
    plsc.subcore_barrier()

    def _dump(c):
        pltpu.sync_copy(acc.at[pl.ds(c * RCHUNK, RCHUNK)],
                        s_out.at[cid, pl.ds(c * RCHUNK, RCHUNK)])

    pl.loop(sid, N // RCHUNK, step=NS)(_dump)


def _final_body(deg_ref, s_ref, g_ref, h_ref, w_ref, o_ref):
    deg = jnp.sum(deg_ref[0], axis=0) + 1.0
    dinv = lax.rsqrt(deg)
    s = s_ref[0] + s_ref[1] + g_ref[...]
    tr = (1.0 - ALPHA) * (s * dinv[:, None]) + ALPHA * h_ref[...]
    o_ref[...] = (1.0 - MIX_B) * tr + MIX_B * jnp.dot(
        tr, w_ref[...], preferred_element_type=jnp.float32)


_final_kernel = pl.pallas_call(
    _final_body,
    grid=(N // BN,),
    in_specs=[
        pl.BlockSpec((1, NW, BN), lambda j: (j, 0, 0)),
        pl.BlockSpec((NC, BN, D), lambda j: (0, j, 0)),
        pl.BlockSpec((BN, D), lambda j: (j, 0)),
        pl.BlockSpec((BN, D), lambda j: (j, 0)),
        pl.BlockSpec((D, D), lambda j: (0, 0)),
    ],
    out_specs=pl.BlockSpec((BN, D), lambda j: (j, 0)),
    out_shape=jax.ShapeDtypeStruct((N, D), jnp.float32),
)


def kernel(features, H0, W, edge_index):
    edges = edge_index.reshape(2 * E)
    deg_p = _deg_kernel(edges)
    deg_t = deg_p.reshape(NW, N // BN, BN).transpose(1, 0, 2)
    g = _scale_kernel(deg_t, features)
    s_p = _agg_kernel(edges, g)
    return _final_kernel(deg_t, s_p, g, H0, W)


# speedup vs baseline: 34.5722x; 1.2066x over previous
import functools
import math

import jax
import jax.numpy as jnp
from jax import lax
from jax.experimental import pallas as pl
from jax.experimental.pallas import tpu as pltpu
from jax.experimental.pallas import tpu_sc as plsc

N = 10000
D = 128
E = 320000
ALPHA = 0.1
MIX_B = math.log1p(0.5 / 3.0)

NC = 2
NS = 16
NW = NC * NS
L = 16

EPW = E // NW
EPC = E // NC
RPS = N // NS
RCHUNK = 80
C = 80
CHUNKS = EPW // C
PAIRS = CHUNKS // 2
LAST = 2 * PAIRS
BN = 1000

_mesh = plsc.VectorSubcoreMesh(core_axis_name="c", subcore_axis_name="s")


@functools.partial(
    pl.kernel,
    out_type=jax.ShapeDtypeStruct((NW * N,), jnp.float32),
    mesh=_mesh,
    scratch_types=[
        pltpu.VMEM((EPW,), jnp.int32),
        pltpu.VMEM((N,), jnp.float32),
    ],
    compiler_params=pltpu.CompilerParams(needs_layout_passes=False),
)
def _deg_kernel(edges_hbm, deg_out, dstbuf, degbuf):
    cid = lax.axis_index("c")
    sid = lax.axis_index("s")
    wid = sid * NC + cid

    zeros = jnp.zeros((L,), jnp.float32)
    ones = jnp.ones((L,), jnp.float32)

    def _zero(i):
        degbuf[pl.ds(i * L, L)] = zeros

    pl.loop(0, N // L)(_zero)

    pltpu.sync_copy(edges_hbm.at[pl.ds(E + wid * EPW, EPW)], dstbuf)

    def _count(i):
        idx = dstbuf[pl.ds(i * L, L)]
        plsc.addupdate_scatter(degbuf, [idx], ones)

    pl.loop(0, EPW // L)(_count)

    pltpu.sync_copy(degbuf, deg_out.at[pl.ds(wid * N, N)])


def _scale_body(deg_ref, f_ref, g_ref):
    deg = jnp.sum(deg_ref[0], axis=0) + 1.0
    dinv = lax.rsqrt(deg)
    g_ref[...] = f_ref[...] * dinv[:, None]


_scale_kernel = pl.pallas_call(
    _scale_body,
    grid=(N // BN,),
    in_specs=[
        pl.BlockSpec((1, NW, BN), lambda j: (j, 0, 0)),
        pl.BlockSpec((BN, D), lambda j: (j, 0)),
    ],
    out_specs=pl.BlockSpec((BN, D), lambda j: (j, 0)),
    out_shape=jax.ShapeDtypeStruct((N, D), jnp.float32),
)


@functools.partial(
    pl.kernel,
    out_type=jax.ShapeDtypeStruct((NC, N, D), jnp.float32),
    mesh=_mesh,
    scratch_types=[
        pltpu.VMEM_SHARED((N, D), jnp.float32),
        pltpu.VMEM((6, C), jnp.int32),
        pltpu.VMEM((6, C), jnp.int32),
        pltpu.VMEM((4 * C, D), jnp.float32),
        pltpu.SemaphoreType.DMA,
        pltpu.SemaphoreType.DMA,
        pltpu.SemaphoreType.DMA,
        pltpu.SemaphoreType.DMA,
    ],
)
def _agg_kernel(edges_hbm, g_hbm, s_out, acc, srcb, dstb,
                rows, gsem, ssem, dsem, asem):
    cid = lax.axis_index("c")
    sid = lax.axis_index("s")
    wid = cid * NS + sid

    zv = jnp.zeros((L,), jnp.float32)

    def _zrow(r):
        def _zcol(j):
            rows[r, pl.ds(j * L, L)] = zv

        pl.loop(0, D // L)(_zcol)

    pl.loop(0, C)(_zrow)

    def _zero(c):
        pltpu.sync_copy(rows.at[pl.ds(0, C)],
                        acc.at[pl.ds(c * RCHUNK, RCHUNK)])

    pl.loop(sid, N // RCHUNK, step=NS)(_zero)

    def _idx_start(t):
        s = t % 6
        off = wid * EPW + t * C
        pltpu.async_copy(edges_hbm.at[pl.ds(off, C)], srcb.at[s], ssem)
        pltpu.async_copy(edges_hbm.at[pl.ds(E + off, C)], dstb.at[s], dsem)

    def _idx_wait(t):
        s = t % 6
        pltpu.make_async_copy(edges_hbm.at[pl.ds(0, C)], srcb.at[s],
                              ssem).wait()
        pltpu.make_async_copy(edges_hbm.at[pl.ds(0, C)], dstb.at[s],
                              dsem).wait()

    _idx_start(0)
    _idx_start(1)
    plsc.subcore_barrier()
    _idx_wait(0)
    _idx_wait(1)
    pltpu.async_copy(g_hbm.at[srcb.at[0]], rows.at[pl.ds(0, C)], gsem)
    pltpu.async_copy(g_hbm.at[srcb.at[1]], rows.at[pl.ds(C, C)], gsem)
    _idx_start(2)
    _idx_start(3)

    def _pair(k):
        b = (k % 2) * (2 * C)
        nb = ((k + 1) % 2) * (2 * C)
        pltpu.make_async_copy(g_hbm.at[pl.ds(0, 2 * C)],
                              rows.at[pl.ds(b, 2 * C)], gsem).wait()

        @pl.when(k > 0)
        def _():
            t0 = 2 * (k - 1)
            pltpu.make_async_copy(rows.at[pl.ds(nb, C)],
                                  acc.at[dstb.at[t0 % 6]], asem).wait()
            pltpu.make_async_copy(rows.at[pl.ds(nb + C, C)],
                                  acc.at[dstb.at[(t0 + 1) % 6]],
                                  asem).wait()

        @pl.when(k + 1 < PAIRS)
        def _():
            t0 = 2 * (k + 1)
            _idx_wait(t0)
            _idx_wait(t0 + 1)
            pltpu.async_copy(g_hbm.at[srcb.at[t0 % 6]],
                             rows.at[pl.ds(nb, C)], gsem)
            pltpu.async_copy(g_hbm.at[srcb.at[(t0 + 1) % 6]],
                             rows.at[pl.ds(nb + C, C)], gsem)

        @pl.when(k + 2 < PAIRS)
        def _():
            t0 = 2 * (k + 2)
            _idx_start(t0)
            _idx_start(t0 + 1)

        pltpu.async_copy(rows.at[pl.ds(b, C)],
                         acc.at[dstb.at[(2 * k) % 6]], asem, add=True)
        pltpu.async_copy(rows.at[pl.ds(b + C, C)],
                         acc.at[dstb.at[(2 * k + 1) % 6]], asem, add=True)

    pl.loop(0, PAIRS)(_pair)

    _idx_start(LAST)
    _idx_wait(LAST)
    pltpu.async_copy(g_hbm.at[srcb.at[LAST % 6]], rows.at[pl.ds(0, C)],
                     gsem)
    pltpu.make_async_copy(g_hbm.at[pl.ds(0, C)], rows.at[pl.ds(0, C)],
                          gsem).wait()
    pltpu.make_async_copy(rows.at[pl.ds(2 * C, C)],
                          acc.at[dstb.at[(2 * (PAIRS - 1)) % 6]],
                          asem).wait()
    pltpu.make_async_copy(rows.at[pl.ds(3 * C, C)],
                          acc.at[dstb.at[(2 * PAIRS - 1) % 6]],
                          asem).wait()
    pltpu.async_copy(rows.at[pl.ds(0, C)], acc.at[dstb.at[LAST % 6]],
                     asem, add=True)
    pltpu.make_async_copy(rows.at[pl.ds(0, C)], acc.at[dstb.at[LAST % 6]],
                          asem).wait()
    plsc.subcore_barrier()

    def _dump(c):
        pltpu.sync_copy(acc.at[pl.ds(c * RCHUNK, RCHUNK)],
                        s_out.at[cid, pl.ds(c * RCHUNK, RCHUNK)])

    pl.loop(sid, N // RCHUNK, step=NS)(_dump)


def _final_body(deg_ref, s_ref, g_ref, h_ref, w_ref, o_ref):
    deg = jnp.sum(deg_ref[0], axis=0) + 1.0
    dinv = lax.rsqrt(deg)
    s = s_ref[0] + s_ref[1] + g_ref[...]
    tr = (1.0 - ALPHA) * (s * dinv[:, None]) + ALPHA * h_ref[...]
    o_ref[...] = (1.0 - MIX_B) * tr + MIX_B * jnp.dot(
        tr, w_ref[...], preferred_element_type=jnp.float32)


_final_kernel = pl.pallas_call(
    _final_body,
    grid=(N // BN,),
    in_specs=[
        pl.BlockSpec((1, NW, BN), lambda j: (j, 0, 0)),
        pl.BlockSpec((NC, BN, D), lambda j: (0, j, 0)),
        pl.BlockSpec((BN, D), lambda j: (j, 0)),
        pl.BlockSpec((BN, D), lambda j: (j, 0)),
        pl.BlockSpec((D, D), lambda j: (0, 0)),
    ],
    out_specs=pl.BlockSpec((BN, D), lambda j: (j, 0)),
    out_shape=jax.ShapeDtypeStruct((N, D), jnp.float32),
)


def kernel(features, H0, W, edge_index):
    edges = edge_index.reshape(2 * E)
    deg_p = _deg_kernel(edges)
    deg_t = deg_p.reshape(NW, N // BN, BN).transpose(1, 0, 2)
    g = _scale_kernel(deg_t, features)
    s_p = _agg_kernel(edges, g)
    return _final_kernel(deg_t, s_p, g, H0, W)


# speedup vs baseline: 40.7073x; 1.1775x over previous
import functools
import math

import jax
import jax.numpy as jnp
from jax import lax
from jax.experimental import pallas as pl
from jax.experimental.pallas import tpu as pltpu
from jax.experimental.pallas import tpu_sc as plsc

N = 10000
D = 128
E = 320000
ALPHA = 0.1
MIX_B = math.log1p(0.5 / 3.0)

NC = 2
NS = 16
NW = NC * NS
L = 16

EPW = E // NW
EPC = E // NC
RPS = N // NS
RCHUNK = 80
C = 80
CHUNKS = EPW // C
PAIRS = CHUNKS // 2
LAST = 2 * PAIRS
BN = 1000

_mesh = plsc.VectorSubcoreMesh(core_axis_name="c", subcore_axis_name="s")


@functools.partial(
    pl.kernel,
    out_type=jax.ShapeDtypeStruct((NW * N,), jnp.float32),
    mesh=_mesh,
    scratch_types=[
        pltpu.VMEM((EPW,), jnp.int32),
        pltpu.VMEM((N,), jnp.float32),
    ],
    compiler_params=pltpu.CompilerParams(needs_layout_passes=False),
)
def _deg_kernel(edges_hbm, deg_out, dstbuf, degbuf):
    cid = lax.axis_index("c")
    sid = lax.axis_index("s")
    wid = sid * NC + cid

    zeros = jnp.zeros((L,), jnp.float32)
    ones = jnp.ones((L,), jnp.float32)

    def _zero(i):
        degbuf[pl.ds(i * L, L)] = zeros

    pl.loop(0, N // L)(_zero)

    pltpu.sync_copy(edges_hbm.at[pl.ds(E + wid * EPW, EPW)], dstbuf)

    def _count(i):
        idx = dstbuf[pl.ds(i * L, L)]
        plsc.addupdate_scatter(degbuf, [idx], ones)

    pl.loop(0, EPW // L)(_count)

    pltpu.sync_copy(degbuf, deg_out.at[pl.ds(wid * N, N)])


def _scale_body(deg_ref, f_ref, g_ref):
    deg = jnp.sum(deg_ref[0], axis=0) + 1.0
    dinv = lax.rsqrt(deg)
    g_ref[...] = f_ref[...] * dinv[:, None]


_scale_kernel = pl.pallas_call(
    _scale_body,
    grid=(N // BN,),
    in_specs=[
        pl.BlockSpec((1, NW, BN), lambda j: (j, 0, 0)),
        pl.BlockSpec((BN, D), lambda j: (j, 0)),
    ],
    out_specs=pl.BlockSpec((BN, D), lambda j: (j, 0)),
    out_shape=jax.ShapeDtypeStruct((N, D), jnp.float32),
)


@functools.partial(
    pl.kernel,
    out_type=jax.ShapeDtypeStruct((NC, N, D), jnp.float32),
    mesh=_mesh,
    scratch_types=[
        pltpu.VMEM_SHARED((N, D), jnp.float32),
        pltpu.VMEM((8, C), jnp.int32),
        pltpu.VMEM((8, C), jnp.int32),
        pltpu.VMEM((4 * C, D), jnp.float32),
        pltpu.SemaphoreType.DMA((4,)),
        pltpu.SemaphoreType.DMA((4,)),
        pltpu.SemaphoreType.DMA((4,)),
        pltpu.SemaphoreType.DMA,
    ],
)
def _agg_kernel(edges_hbm, g_hbm, s_out, acc, srcb, dstb,
                rows, gsems, ssems, dsems, asem):
    cid = lax.axis_index("c")
    sid = lax.axis_index("s")
    wid = cid * NS + sid

    zv = jnp.zeros((L,), jnp.float32)

    def _zrow(r):
        def _zcol(j):
            rows[r, pl.ds(j * L, L)] = zv

        pl.loop(0, D // L)(_zcol)

    pl.loop(0, C)(_zrow)

    def _zero(c):
        pltpu.sync_copy(rows.at[pl.ds(0, C)],
                        acc.at[pl.ds(c * RCHUNK, RCHUNK)])

    pl.loop(sid, N // RCHUNK, step=NS)(_zero)

    def _idx_start(t):
        s = t % 8
        off = wid * EPW + t * C
        pltpu.async_copy(edges_hbm.at[pl.ds(off, C)], srcb.at[s],
                         ssems.at[t % 4])
        pltpu.async_copy(edges_hbm.at[pl.ds(E + off, C)], dstb.at[s],
                         dsems.at[t % 4])

    def _idx_wait(t):
        s = t % 8
        pltpu.make_async_copy(edges_hbm.at[pl.ds(0, C)], srcb.at[s],
                              ssems.at[t % 4]).wait()
        pltpu.make_async_copy(edges_hbm.at[pl.ds(0, C)], dstb.at[s],
                              dsems.at[t % 4]).wait()

    def _gather_start(t):
        pltpu.async_copy(g_hbm.at[srcb.at[t % 8]],
                         rows.at[pl.ds((t % 4) * C, C)], gsems.at[t % 4])

    def _gather_wait(t):
        pltpu.make_async_copy(g_hbm.at[pl.ds(0, C)],
                              rows.at[pl.ds((t % 4) * C, C)],
                              gsems.at[t % 4]).wait()

    def _scatter_wait(t):
        pltpu.make_async_copy(rows.at[pl.ds((t % 4) * C, C)],
                              acc.at[dstb.at[t % 8]], asem).wait()

    _idx_start(0)
    _idx_start(1)
    _idx_start(2)
    _idx_start(3)
    plsc.subcore_barrier()
    for t in range(3):
        _idx_wait(t)
        _gather_start(t)

    def _step(t):
        _gather_wait(t)

        @pl.when(t > 0)
        def _():
            _scatter_wait(t - 1)

        @pl.when(t + 3 < CHUNKS)
        def _():
            _idx_wait(t + 3)
            _gather_start(t + 3)

        @pl.when(t + 4 < CHUNKS)
        def _():
            _idx_start(t + 4)

        pltpu.async_copy(rows.at[pl.ds((t % 4) * C, C)],
                         acc.at[dstb.at[t % 8]], asem, add=True)

    pl.loop(0, CHUNKS)(_step)

    _scatter_wait(CHUNKS - 1)
    plsc.subcore_barrier()

    def _dump(c):
        pltpu.sync_copy(acc.at[pl.ds(c * RCHUNK, RCHUNK)],
                        s_out.at[cid, pl.ds(c * RCHUNK, RCHUNK)])

    pl.loop(sid, N // RCHUNK, step=NS)(_dump)


def _final_body(deg_ref, s_ref, g_ref, h_ref, w_ref, o_ref):
    deg = jnp.sum(deg_ref[0], axis=0) + 1.0
    dinv = lax.rsqrt(deg)
    s = s_ref[0] + s_ref[1] + g_ref[...]
    tr = (1.0 - ALPHA) * (s * dinv[:, None]) + ALPHA * h_ref[...]
    o_ref[...] = (1.0 - MIX_B) * tr + MIX_B * jnp.dot(
        tr, w_ref[...], preferred_element_type=jnp.float32)


_final_kernel = pl.pallas_call(
    _final_body,
    grid=(N // BN,),
    in_specs=[
        pl.BlockSpec((1, NW, BN), lambda j: (j, 0, 0)),
        pl.BlockSpec((NC, BN, D), lambda j: (0, j, 0)),
        pl.BlockSpec((BN, D), lambda j: (j, 0)),
        pl.BlockSpec((BN, D), lambda j: (j, 0)),
        pl.BlockSpec((D, D), lambda j: (0, 0)),
    ],
    out_specs=pl.BlockSpec((BN, D), lambda j: (j, 0)),
    out_shape=jax.ShapeDtypeStruct((N, D), jnp.float32),
)


def kernel(features, H0, W, edge_index):
    edges = edge_index.reshape(2 * E)
    deg_p = _deg_kernel(edges)
    deg_t = deg_p.reshape(NW, N // BN, BN).transpose(1, 0, 2)
    g = _scale_kernel(deg_t, features)
    s_p = _agg_kernel(edges, g)
    return _final_kernel(deg_t, s_p, g, H0, W)
